# Initial kernel scaffold; baseline (speedup 1.0000x reference)
import functools

import jax
import jax.numpy as jnp
from jax import lax
from jax.experimental import pallas as pl
from jax.experimental.pallas import tpu as pltpu
from jax.experimental.pallas import tpu_sc as plsc

N = 10000
D = 128
H = 128
OUT = 64
CH = 128
NW = 32
NACC = 10016
RPT = NACC // 16
RB = 1000

_MESH = plsc.VectorSubcoreMesh(core_axis_name="c", subcore_axis_name="s")


def _hop0_agg(table, srcs, dsts0, dsts1, zfeat, zcnt, ones, K):
    @functools.partial(
        pl.kernel,
        out_type=[
            jax.ShapeDtypeStruct((2, NACC, H), jnp.float32),
            jax.ShapeDtypeStruct((2, NACC, 16), jnp.float32),
            jax.ShapeDtypeStruct((2, NACC, 16), jnp.float32),
        ],
        mesh=_MESH,
        scratch_types=[
            pltpu.VMEM((CH, H), jnp.float32),
            pltpu.VMEM((CH, 16), jnp.float32),
            pltpu.VMEM((K, CH), jnp.int32),
            pltpu.VMEM((K, CH), jnp.int32),
            pltpu.VMEM((K, CH), jnp.int32),
            pltpu.VMEM_SHARED((NACC, H), jnp.float32),
            pltpu.VMEM_SHARED((NACC, 16), jnp.float32),
            pltpu.VMEM_SHARED((NACC, 16), jnp.float32),
        ],
    )
    def k(table_h, srcs_h, dsts0_h, dsts1_h, zf_h, zc_h, ones_h,
          ofeat_h, ocnt0_h, ocnt1_h,
          rows_v, ones_v, src_v, dst0_v, dst1_v, acc_s, c0_s, c1_s):
        c = lax.axis_index("c")
        s = lax.axis_index("s")
        w = c * 16 + s
        r0 = s * RPT
        pltpu.sync_copy(zf_h.at[pl.ds(r0, RPT)], acc_s.at[pl.ds(r0, RPT)])
        pltpu.sync_copy(zc_h.at[pl.ds(r0, RPT)], c0_s.at[pl.ds(r0, RPT)])
        pltpu.sync_copy(zc_h.at[pl.ds(r0, RPT)], c1_s.at[pl.ds(r0, RPT)])
        pltpu.sync_copy(ones_h, ones_v)
        pltpu.sync_copy(srcs_h.at[w], src_v)
        pltpu.sync_copy(dsts0_h.at[w], dst0_v)
        pltpu.sync_copy(dsts1_h.at[w], dst1_v)
        plsc.subcore_barrier()

        @pl.loop(0, K)
        def _(j):
            pltpu.sync_copy(table_h.at[src_v.at[j]], rows_v)
            pltpu.sync_copy(rows_v, acc_s.at[dst0_v.at[j]], add=True)
            pltpu.sync_copy(ones_v, c0_s.at[dst0_v.at[j]], add=True)
            pltpu.sync_copy(ones_v, c1_s.at[dst1_v.at[j]], add=True)

        plsc.subcore_barrier()
        pltpu.sync_copy(acc_s.at[pl.ds(r0, RPT)], ofeat_h.at[c, pl.ds(r0, RPT)])
        pltpu.sync_copy(c0_s.at[pl.ds(r0, RPT)], ocnt0_h.at[c, pl.ds(r0, RPT)])
        pltpu.sync_copy(c1_s.at[pl.ds(r0, RPT)], ocnt1_h.at[c, pl.ds(r0, RPT)])

    return k(table, srcs, dsts0, dsts1, zfeat, zcnt, ones)


def _hop1_agg(table, srcs, dsts, zfeat, K):
    @functools.partial(
        pl.kernel,
        out_type=jax.ShapeDtypeStruct((2, NACC, OUT), jnp.float32),
        mesh=_MESH,
        scratch_types=[
            pltpu.VMEM((CH, OUT), jnp.float32),
            pltpu.VMEM((K, CH), jnp.int32),
            pltpu.VMEM((K, CH), jnp.int32),
            pltpu.VMEM_SHARED((NACC, OUT), jnp.float32),
        ],
    )
    def k(table_h, srcs_h, dsts_h, zf_h, ofeat_h,
          rows_v, src_v, dst_v, acc_s):
        c = lax.axis_index("c")
        s = lax.axis_index("s")
        w = c * 16 + s
        r0 = s * RPT
        pltpu.sync_copy(zf_h.at[pl.ds(r0, RPT)], acc_s.at[pl.ds(r0, RPT)])
        pltpu.sync_copy(srcs_h.at[w], src_v)
        pltpu.sync_copy(dsts_h.at[w], dst_v)
        plsc.subcore_barrier()

        @pl.loop(0, K)
        def _(j):
            pltpu.sync_copy(table_h.at[src_v.at[j]], rows_v)
            pltpu.sync_copy(rows_v, acc_s.at[dst_v.at[j]], add=True)

        plsc.subcore_barrier()
        pltpu.sync_copy(acc_s.at[pl.ds(r0, RPT)], ofeat_h.at[c, pl.ds(r0, RPT)])

    return k(table, srcs, dsts, zfeat)


def _proj_body(x_ref, w_ref, b_ref, o_ref):
    o_ref[...] = jnp.maximum(
        jnp.dot(x_ref[...], w_ref[...], preferred_element_type=jnp.float32)
        + b_ref[...], 0.0)


def _proj(x, wT, b):
    return pl.pallas_call(
        _proj_body,
        grid=(N // RB,),
        in_specs=[
            pl.BlockSpec((RB, D), lambda i: (i, 0)),
            pl.BlockSpec((D, H), lambda i: (0, 0)),
            pl.BlockSpec((1, H), lambda i: (0, 0)),
        ],
        out_specs=pl.BlockSpec((RB, H), lambda i: (i, 0)),
        out_shape=jax.ShapeDtypeStruct((N, H), jnp.float32),
    )(x, wT, b)


def _merge0_body(f_ref, c_ref, hp_ref, ha_ref, w0l_ref, b0_ref, w0r_ref,
                 w1l_ref, w1r_ref, b1_ref, p1_ref, r1_ref):
    cnt = jnp.maximum(c_ref[0, :, 0:1] + c_ref[1, :, 0:1], 1.0)
    mean = (f_ref[0] + f_ref[1]) / cnt
    t = (jnp.dot(mean, w0l_ref[...], preferred_element_type=jnp.float32)
         + b0_ref[...]
         + jnp.dot(hp_ref[...], w0r_ref[...], preferred_element_type=jnp.float32))
    h = jnp.maximum(t, 0.0)
    p1_ref[...] = jnp.dot(h, w1l_ref[...], preferred_element_type=jnp.float32)
    r1_ref[...] = (jnp.dot(ha_ref[...], w1r_ref[...],
                           preferred_element_type=jnp.float32) + b1_ref[...])


def _merge0(feat, cnt, h_paper, h_author, w0lT, b0, w0rT, w1lT, w1rT, b1):
    return pl.pallas_call(
        _merge0_body,
        grid=(N // RB,),
        in_specs=[
            pl.BlockSpec((2, RB, H), lambda i: (0, i, 0)),
            pl.BlockSpec((2, RB, 16), lambda i: (0, i, 0)),
            pl.BlockSpec((RB, H), lambda i: (i, 0)),
            pl.BlockSpec((RB, H), lambda i: (i, 0)),
            pl.BlockSpec((H, H), lambda i: (0, 0)),
            pl.BlockSpec((1, H), lambda i: (0, 0)),
            pl.BlockSpec((H, H), lambda i: (0, 0)),
            pl.BlockSpec((H, OUT), lambda i: (0, 0)),
            pl.BlockSpec((H, OUT), lambda i: (0, 0)),
            pl.BlockSpec((1, OUT), lambda i: (0, 0)),
        ],
        out_specs=[
            pl.BlockSpec((RB, OUT), lambda i: (i, 0)),
            pl.BlockSpec((RB, OUT), lambda i: (i, 0)),
        ],
        out_shape=[
            jax.ShapeDtypeStruct((N, OUT), jnp.float32),
            jax.ShapeDtypeStruct((N, OUT), jnp.float32),
        ],
    )(feat, cnt, h_paper, h_author, w0lT, b0, w0rT, w1lT, w1rT, b1)


def _final_body(f_ref, c_ref, r1_ref, o_ref):
    cnt = jnp.maximum(c_ref[0, :, 0:1] + c_ref[1, :, 0:1], 1.0)
    o_ref[...] = (f_ref[0] + f_ref[1]) / cnt + r1_ref[...]


def _final(feat, cnt, r1):
    return pl.pallas_call(
        _final_body,
        grid=(N // RB,),
        in_specs=[
            pl.BlockSpec((2, RB, OUT), lambda i: (0, i, 0)),
            pl.BlockSpec((2, RB, 16), lambda i: (0, i, 0)),
            pl.BlockSpec((RB, OUT), lambda i: (i, 0)),
        ],
        out_specs=pl.BlockSpec((RB, OUT), lambda i: (i, 0)),
        out_shape=jax.ShapeDtypeStruct((N, OUT), jnp.float32),
    )(feat, cnt, r1)


def _pad_edges(idx, fill, K):
    pad = NW * K * CH - idx.shape[0]
    return jnp.concatenate(
        [idx.astype(jnp.int32),
         jnp.full((pad,), fill, jnp.int32)]).reshape(NW, K, CH)


def kernel(x_author, x_paper, edge_index_hop0, edge_index_hop1,
           W_proj_author, b_proj_author, W_proj_paper, b_proj_paper,
           W0_l, b0_l, W0_r, W1_l, b1_l, W1_r):
    E = edge_index_hop0.shape[1]
    K = -(-E // (NW * CH))

    src0 = _pad_edges(edge_index_hop0[0], 0, K)
    dst0 = _pad_edges(edge_index_hop0[1], N, K)
    src1 = _pad_edges(edge_index_hop1[0], 0, K)
    dst1 = _pad_edges(edge_index_hop1[1], N, K)

    zfeat = jnp.zeros((NACC, H), jnp.float32)
    zfeat1 = jnp.zeros((NACC, OUT), jnp.float32)
    zcnt = jnp.zeros((NACC, 16), jnp.float32)
    ones = jnp.ones((CH, 16), jnp.float32)

    h_author = _proj(x_author, W_proj_author.T, b_proj_author.reshape(1, H))
    h_paper = _proj(x_paper, W_proj_paper.T, b_proj_paper.reshape(1, H))

    feat0, cnt0, cnt1 = _hop0_agg(h_author, src0, dst0, dst1,
                                  zfeat, zcnt, ones, K)

    p1, r1 = _merge0(feat0, cnt0, h_paper, h_author,
                     W0_l.T, b0_l.reshape(1, H), W0_r.T,
                     W1_l.T, W1_r.T, b1_l.reshape(1, OUT))

    feat1 = _hop1_agg(p1, src1, dst1, zfeat1, K)

    return _final(feat1, cnt1, r1)


# speedup vs baseline: 6.1066x; 6.1066x over previous
import functools

import jax
import jax.numpy as jnp
from jax import lax
from jax.experimental import pallas as pl
from jax.experimental.pallas import tpu as pltpu
from jax.experimental.pallas import tpu_sc as plsc

N = 10000
D = 128
H = 128
OUT = 64
CH = 128
NW = 32
NACC = 10240
RPT = NACC // 16
NB = RPT // CH
GRP = 8
NG = 10
RB = 1024
GRID = 10

_MESH = plsc.VectorSubcoreMesh(core_axis_name="c", subcore_axis_name="s")


def _hop_agg(table, srcs, dsts, zrow):
    @functools.partial(
        pl.kernel,
        out_type=jax.ShapeDtypeStruct((2, NACC, H), jnp.float32),
        mesh=_MESH,
        scratch_types=[
            pltpu.VMEM((CH, H), jnp.float32),
            pltpu.VMEM((GRP, CH), jnp.int32),
            pltpu.VMEM((GRP, CH), jnp.int32),
            pltpu.VMEM_SHARED((NACC, H), jnp.float32),
        ],
    )
    def k(table_h, srcs_h, dsts_h, zrow_h, ofeat_h,
          rows_v, src_v, dst_v, acc_s):
        c = lax.axis_index("c")
        s = lax.axis_index("s")
        w = c * 16 + s
        r0 = s * RPT
        pltpu.sync_copy(zrow_h, rows_v)
        for t in range(NB):
            pltpu.sync_copy(rows_v, acc_s.at[pl.ds(r0 + t * CH, CH)])
        plsc.subcore_barrier()

        @pl.loop(0, NG)
        def _(g):
            pltpu.sync_copy(srcs_h.at[w * NG + g], src_v)
            pltpu.sync_copy(dsts_h.at[w * NG + g], dst_v)

            @pl.loop(0, GRP)
            def _(j):
                pltpu.sync_copy(table_h.at[src_v.at[j]], rows_v)
                pltpu.sync_copy(rows_v, acc_s.at[dst_v.at[j]], add=True)

        plsc.subcore_barrier()
        for t in range(NB):
            pltpu.sync_copy(acc_s.at[pl.ds(r0 + t * CH, CH)], rows_v)
            pltpu.sync_copy(rows_v, ofeat_h.at[c, pl.ds(r0 + t * CH, CH)])

    return k(table, srcs, dsts, zrow)


def _counts(dsts0, dsts1, zrow, onerow):
    @functools.partial(
        pl.kernel,
        out_type=[
            jax.ShapeDtypeStruct((2, NACC, H), jnp.float32),
            jax.ShapeDtypeStruct((2, NACC, H), jnp.float32),
        ],
        mesh=_MESH,
        scratch_types=[
            pltpu.VMEM((CH, H), jnp.float32),
            pltpu.VMEM((GRP, CH), jnp.int32),
            pltpu.VMEM_SHARED((NACC, H), jnp.float32),
        ],
    )
    def k(dsts0_h, dsts1_h, zrow_h, onerow_h, oc0_h, oc1_h,
          rows_v, dst_v, acc_s):
        c = lax.axis_index("c")
        s = lax.axis_index("s")
        w = c * 16 + s
        r0 = s * RPT

        def one_hop(dsts_h, out_h):
            pltpu.sync_copy(zrow_h, rows_v)
            for t in range(NB):
                pltpu.sync_copy(rows_v, acc_s.at[pl.ds(r0 + t * CH, CH)])
            pltpu.sync_copy(onerow_h, rows_v)
            plsc.subcore_barrier()

            @pl.loop(0, NG)
            def _(g):
                pltpu.sync_copy(dsts_h.at[w * NG + g], dst_v)

                @pl.loop(0, GRP)
                def _(j):
                    pltpu.sync_copy(rows_v, acc_s.at[dst_v.at[j]], add=True)

            plsc.subcore_barrier()
            for t in range(NB):
                pltpu.sync_copy(acc_s.at[pl.ds(r0 + t * CH, CH)], rows_v)
                pltpu.sync_copy(rows_v, out_h.at[c, pl.ds(r0 + t * CH, CH)])
            plsc.subcore_barrier()

        one_hop(dsts0_h, oc0_h)
        one_hop(dsts1_h, oc1_h)

    return k(dsts0, dsts1, zrow, onerow)


def _proj_body(x_ref, w_ref, b_ref, o_ref):
    o_ref[...] = jnp.maximum(
        jnp.dot(x_ref[...], w_ref[...], preferred_element_type=jnp.float32)
        + b_ref[...], 0.0)


def _proj(x, wT, b):
    return pl.pallas_call(
        _proj_body,
        grid=(GRID,),
        in_specs=[
            pl.BlockSpec((RB, D), lambda i: (i, 0)),
            pl.BlockSpec((D, H), lambda i: (0, 0)),
            pl.BlockSpec((1, H), lambda i: (0, 0)),
        ],
        out_specs=pl.BlockSpec((RB, H), lambda i: (i, 0)),
        out_shape=jax.ShapeDtypeStruct((N, H), jnp.float32),
    )(x, wT, b)


def _merge0_body(f_ref, c_ref, hp_ref, ha_ref, w0l_ref, b0_ref, w0r_ref,
                 w1r_ref, b1_ref, h_ref, r1_ref):
    cnt = jnp.maximum(c_ref[0, :, 0:1] + c_ref[1, :, 0:1], 1.0)
    mean = (f_ref[0] + f_ref[1]) / cnt
    t = (jnp.dot(mean, w0l_ref[...], preferred_element_type=jnp.float32)
         + b0_ref[...]
         + jnp.dot(hp_ref[...], w0r_ref[...], preferred_element_type=jnp.float32))
    h_ref[...] = jnp.maximum(t, 0.0)
    r1_ref[...] = (jnp.dot(ha_ref[...], w1r_ref[...],
                           preferred_element_type=jnp.float32) + b1_ref[...])


def _merge0(feat, cnt, h_paper, h_author, w0lT, b0, w0rT, w1rT, b1):
    return pl.pallas_call(
        _merge0_body,
        grid=(GRID,),
        in_specs=[
            pl.BlockSpec((2, RB, H), lambda i: (0, i, 0)),
            pl.BlockSpec((2, RB, H), lambda i: (0, i, 0)),
            pl.BlockSpec((RB, H), lambda i: (i, 0)),
            pl.BlockSpec((RB, H), lambda i: (i, 0)),
            pl.BlockSpec((H, H), lambda i: (0, 0)),
            pl.BlockSpec((1, H), lambda i: (0, 0)),
            pl.BlockSpec((H, H), lambda i: (0, 0)),
            pl.BlockSpec((H, OUT), lambda i: (0, 0)),
            pl.BlockSpec((1, OUT), lambda i: (0, 0)),
        ],
        out_specs=[
            pl.BlockSpec((RB, H), lambda i: (i, 0)),
            pl.BlockSpec((RB, OUT), lambda i: (i, 0)),
        ],
        out_shape=[
            jax.ShapeDtypeStruct((N, H), jnp.float32),
            jax.ShapeDtypeStruct((N, OUT), jnp.float32),
        ],
    )(feat, cnt, h_paper, h_author, w0lT, b0, w0rT, w1rT, b1)


def _final_body(f_ref, c_ref, r1_ref, w1l_ref, o_ref):
    cnt = jnp.maximum(c_ref[0, :, 0:1] + c_ref[1, :, 0:1], 1.0)
    mean = (f_ref[0] + f_ref[1]) / cnt
    o_ref[...] = (jnp.dot(mean, w1l_ref[...],
                          preferred_element_type=jnp.float32) + r1_ref[...])


def _final(feat, cnt, r1, w1lT):
    return pl.pallas_call(
        _final_body,
        grid=(GRID,),
        in_specs=[
            pl.BlockSpec((2, RB, H), lambda i: (0, i, 0)),
            pl.BlockSpec((2, RB, H), lambda i: (0, i, 0)),
            pl.BlockSpec((RB, OUT), lambda i: (i, 0)),
            pl.BlockSpec((H, OUT), lambda i: (0, 0)),
        ],
        out_specs=pl.BlockSpec((RB, OUT), lambda i: (i, 0)),
        out_shape=jax.ShapeDtypeStruct((N, OUT), jnp.float32),
    )(feat, cnt, r1, w1lT)


def kernel(x_author, x_paper, edge_index_hop0, edge_index_hop1,
           W_proj_author, b_proj_author, W_proj_paper, b_proj_paper,
           W0_l, b0_l, W0_r, W1_l, b1_l, W1_r):
    E = edge_index_hop0.shape[1]
    EP = NW * NG * GRP * CH
    npad = EP - E
    pad_src = (jnp.arange(npad, dtype=jnp.int32) * 37) % N
    pad_dst = N + (jnp.arange(npad, dtype=jnp.int32) % (NACC - N))

    def prep(idx, pad_vals):
        return jnp.concatenate(
            [idx.astype(jnp.int32), pad_vals]).reshape(NW * NG, GRP, CH)

    src0 = prep(edge_index_hop0[0], pad_src)
    dst0 = prep(edge_index_hop0[1], pad_dst)
    src1 = prep(edge_index_hop1[0], pad_src)
    dst1 = prep(edge_index_hop1[1], pad_dst)

    zrow = jnp.zeros((CH, H), jnp.float32)
    onerow = jnp.ones((CH, H), jnp.float32)

    h_author = _proj(x_author, W_proj_author.T, b_proj_author.reshape(1, H))
    h_paper = _proj(x_paper, W_proj_paper.T, b_proj_paper.reshape(1, H))

    cnt0, cnt1 = _counts(dst0, dst1, zrow, onerow)

    feat0 = _hop_agg(h_author, src0, dst0, zrow)

    h, r1 = _merge0(feat0, cnt0, h_paper, h_author,
                    W0_l.T, b0_l.reshape(1, H), W0_r.T,
                    W1_r.T, b1_l.reshape(1, OUT))

    feat1 = _hop_agg(h, src1, dst1, zrow)

    return _final(feat1, cnt1, r1, W1_l.T)


# speedup vs baseline: 6.9224x; 1.1336x over previous
import functools

import jax
import jax.numpy as jnp
from jax import lax
from jax.experimental import pallas as pl
from jax.experimental.pallas import tpu as pltpu
from jax.experimental.pallas import tpu_sc as plsc

N = 10000
D = 128
H = 128
OUT = 64
CH = 128
NW = 32
NACC = 10240
RPT = NACC // 16
NB = RPT // CH
GRP = 8
NG = 10
RB = 1024
GRID = 10

_MESH = plsc.VectorSubcoreMesh(core_axis_name="c", subcore_axis_name="s")


def _hop_agg(table, srcs, dsts, zrow):
    @functools.partial(
        pl.kernel,
        out_type=jax.ShapeDtypeStruct((2, NACC, H), jnp.float32),
        mesh=_MESH,
        scratch_types=[
            pltpu.VMEM((CH, H), jnp.float32),
            pltpu.VMEM((GRP, CH), jnp.int32),
            pltpu.VMEM((GRP, CH), jnp.int32),
            pltpu.VMEM_SHARED((NACC, H), jnp.float32),
        ],
    )
    def k(table_h, srcs_h, dsts_h, zrow_h, ofeat_h,
          rows_v, src_v, dst_v, acc_s):
        c = lax.axis_index("c")
        s = lax.axis_index("s")
        w = c * 16 + s
        r0 = s * RPT
        pltpu.sync_copy(zrow_h, rows_v)
        for t in range(NB):
            pltpu.sync_copy(rows_v, acc_s.at[pl.ds(r0 + t * CH, CH)])
        plsc.subcore_barrier()

        @pl.loop(0, NG)
        def _(g):
            pltpu.sync_copy(srcs_h.at[w * NG + g], src_v)
            pltpu.sync_copy(dsts_h.at[w * NG + g], dst_v)

            @pl.loop(0, GRP)
            def _(j):
                pltpu.sync_copy(table_h.at[src_v.at[j]], rows_v)
                pltpu.sync_copy(rows_v, acc_s.at[dst_v.at[j]], add=True)

        plsc.subcore_barrier()
        for t in range(NB):
            pltpu.sync_copy(acc_s.at[pl.ds(r0 + t * CH, CH)], rows_v)
            pltpu.sync_copy(rows_v, ofeat_h.at[c, pl.ds(r0 + t * CH, CH)])

    return k(table, srcs, dsts, zrow)


def _cnt_body(d_ref, o_ref):
    d = d_ref[...]
    q = jax.lax.shift_right_logical(d, 7)
    r = jax.lax.bitwise_and(d, 127)
    lanes = jax.lax.broadcasted_iota(jnp.int32, (1, H), 1)
    a = jnp.where(q == lanes, 1.0, 0.0)
    b = jnp.where(r == lanes, 1.0, 0.0)
    part = jax.lax.dot_general(a, b, (((0,), (0,)), ((), ())),
                               preferred_element_type=jnp.float32)

    @pl.when(pl.program_id(0) == 0)
    def _():
        o_ref[...] = jnp.zeros_like(o_ref)

    o_ref[...] += part


EC = 8192


def _cnt_tc(dst_col):
    return pl.pallas_call(
        _cnt_body,
        grid=(dst_col.shape[0] // EC,),
        in_specs=[pl.BlockSpec((EC, 1), lambda i: (i, 0))],
        out_specs=pl.BlockSpec((H, H), lambda i: (0, 0)),
        out_shape=jax.ShapeDtypeStruct((H, H), jnp.float32),
    )(dst_col)


def _cnt_col(c_ref):
    m = c_ref[...]
    i0 = jax.lax.broadcasted_iota(jnp.int32, (RB, 8), 0) // H
    s0 = jax.lax.broadcasted_iota(jnp.int32, (RB, 8), 1)
    p = jnp.where(i0 == s0, 1.0, 0.0)
    y = jnp.dot(p, m, preferred_element_type=jnp.float32)
    i1 = jax.lax.broadcasted_iota(jnp.int32, (RB, H), 0) % H
    t1 = jax.lax.broadcasted_iota(jnp.int32, (RB, H), 1)
    qm = jnp.where(i1 == t1, 1.0, 0.0)
    return jnp.maximum(jnp.sum(y * qm, axis=1, keepdims=True), 1.0)


def _proj_body(x_ref, w_ref, b_ref, o_ref):
    o_ref[...] = jnp.maximum(
        jnp.dot(x_ref[...], w_ref[...], preferred_element_type=jnp.float32)
        + b_ref[...], 0.0)


def _proj(x, wT, b):
    return pl.pallas_call(
        _proj_body,
        grid=(GRID,),
        in_specs=[
            pl.BlockSpec((RB, D), lambda i: (i, 0)),
            pl.BlockSpec((D, H), lambda i: (0, 0)),
            pl.BlockSpec((1, H), lambda i: (0, 0)),
        ],
        out_specs=pl.BlockSpec((RB, H), lambda i: (i, 0)),
        out_shape=jax.ShapeDtypeStruct((N, H), jnp.float32),
    )(x, wT, b)


def _merge0_body(f_ref, c_ref, hp_ref, ha_ref, w0l_ref, b0_ref, w0r_ref,
                 w1r_ref, b1_ref, h_ref, r1_ref):
    mean = (f_ref[0] + f_ref[1]) / _cnt_col(c_ref)
    t = (jnp.dot(mean, w0l_ref[...], preferred_element_type=jnp.float32)
         + b0_ref[...]
         + jnp.dot(hp_ref[...], w0r_ref[...], preferred_element_type=jnp.float32))
    h_ref[...] = jnp.maximum(t, 0.0)
    r1_ref[...] = (jnp.dot(ha_ref[...], w1r_ref[...],
                           preferred_element_type=jnp.float32) + b1_ref[...])


def _merge0(feat, cnt, h_paper, h_author, w0lT, b0, w0rT, w1rT, b1):
    return pl.pallas_call(
        _merge0_body,
        grid=(GRID,),
        in_specs=[
            pl.BlockSpec((2, RB, H), lambda i: (0, i, 0)),
            pl.BlockSpec((8, H), lambda i: (i, 0)),
            pl.BlockSpec((RB, H), lambda i: (i, 0)),
            pl.BlockSpec((RB, H), lambda i: (i, 0)),
            pl.BlockSpec((H, H), lambda i: (0, 0)),
            pl.BlockSpec((1, H), lambda i: (0, 0)),
            pl.BlockSpec((H, H), lambda i: (0, 0)),
            pl.BlockSpec((H, OUT), lambda i: (0, 0)),
            pl.BlockSpec((1, OUT), lambda i: (0, 0)),
        ],
        out_specs=[
            pl.BlockSpec((RB, H), lambda i: (i, 0)),
            pl.BlockSpec((RB, OUT), lambda i: (i, 0)),
        ],
        out_shape=[
            jax.ShapeDtypeStruct((N, H), jnp.float32),
            jax.ShapeDtypeStruct((N, OUT), jnp.float32),
        ],
    )(feat, cnt, h_paper, h_author, w0lT, b0, w0rT, w1rT, b1)


def _final_body(f_ref, c_ref, r1_ref, w1l_ref, o_ref):
    mean = (f_ref[0] + f_ref[1]) / _cnt_col(c_ref)
    o_ref[...] = (jnp.dot(mean, w1l_ref[...],
                          preferred_element_type=jnp.float32) + r1_ref[...])


def _final(feat, cnt, r1, w1lT):
    return pl.pallas_call(
        _final_body,
        grid=(GRID,),
        in_specs=[
            pl.BlockSpec((2, RB, H), lambda i: (0, i, 0)),
            pl.BlockSpec((8, H), lambda i: (i, 0)),
            pl.BlockSpec((RB, OUT), lambda i: (i, 0)),
            pl.BlockSpec((H, OUT), lambda i: (0, 0)),
        ],
        out_specs=pl.BlockSpec((RB, OUT), lambda i: (i, 0)),
        out_shape=jax.ShapeDtypeStruct((N, OUT), jnp.float32),
    )(feat, cnt, r1, w1lT)


def kernel(x_author, x_paper, edge_index_hop0, edge_index_hop1,
           W_proj_author, b_proj_author, W_proj_paper, b_proj_paper,
           W0_l, b0_l, W0_r, W1_l, b1_l, W1_r):
    E = edge_index_hop0.shape[1]
    EP = NW * NG * GRP * CH
    npad = EP - E
    pad_src = (jnp.arange(npad, dtype=jnp.int32) * 37) % N
    pad_dst = N + (jnp.arange(npad, dtype=jnp.int32) % (NACC - N))

    def prep(idx, pad_vals):
        flat = jnp.concatenate([idx.astype(jnp.int32), pad_vals])
        return flat.reshape(NW * NG, GRP, CH), flat.reshape(EP, 1)

    src0, _ = prep(edge_index_hop0[0], pad_src)
    dst0, dcol0 = prep(edge_index_hop0[1], pad_dst)
    src1, _ = prep(edge_index_hop1[0], pad_src)
    dst1, dcol1 = prep(edge_index_hop1[1], pad_dst)

    zrow = jnp.zeros((CH, H), jnp.float32)

    h_author = _proj(x_author, W_proj_author.T, b_proj_author.reshape(1, H))
    h_paper = _proj(x_paper, W_proj_paper.T, b_proj_paper.reshape(1, H))

    cnt0 = _cnt_tc(dcol0)
    cnt1 = _cnt_tc(dcol1)

    feat0 = _hop_agg(h_author, src0, dst0, zrow)

    h, r1 = _merge0(feat0, cnt0, h_paper, h_author,
                    W0_l.T, b0_l.reshape(1, H), W0_r.T,
                    W1_r.T, b1_l.reshape(1, OUT))

    feat1 = _hop_agg(h, src1, dst1, zrow)

    return _final(feat1, cnt1, r1, W1_l.T)
